# baseline (speedup 1.0000x reference)
import functools

import jax
import jax.numpy as jnp
from jax import lax
from jax.experimental import pallas as pl
from jax.experimental.pallas import tpu as pltpu
from jax.experimental.pallas import tpu_sc as plsc

B = 4
S = 8192
D = 768
K = 2048
LOG2S = 13


R = 8
C = S // R


def _sort_body(key_ref, gidx_ref):
    key = key_ref[...]
    nr = key_ref.shape[0]
    g = lax.broadcasted_iota(jnp.int32, (nr, C), 0)
    cc = lax.broadcasted_iota(jnp.int32, (nr, C), 1)
    it = cc * R + (g & (R - 1))
    idx = (g & (R - 1)) * C + cc

    for klog in range(1, LOG2S + 1):
        kk = 1 << klog
        for jlog in range(klog - 1, -1, -1):
            j = 1 << jlog
            is_hi = (it & j) != 0
            dir_up = (it & kk) == 0

            def partner(x, j=j, is_hi=is_hi, nr=nr):
                if j < R:
                    return jnp.where(is_hi, pltpu.roll(x, j, 0),
                                     pltpu.roll(x, nr - j, 0))
                d = j // R
                return jnp.where(is_hi, pltpu.roll(x, d, 1),
                                 pltpu.roll(x, C - d, 1))

            pk = partner(key)
            pi = partner(idx)
            less = (key > pk) | ((key == pk) & (idx < pi))
            keep = jnp.logical_xor(less, is_hi) == dir_up
            key = jnp.where(keep, key, pk)
            idx = jnp.where(keep, idx, pi)

    gidx_ref[...] = idx + (g >> 3) * S


_BH = 2
_sort_call = pl.pallas_call(
    _sort_body,
    out_shape=jax.ShapeDtypeStruct((_BH * R, C), jnp.int32),
)


_NC, _NS = 2, 16
_NW = _NC * _NS
_RPW = (_BH * K) // _NW
_CHUNK = 32
_NBUF = 4
_NCH = _RPW // _CHUNK

_PPW = (_BH * S) // _NW
_WPR = _NW // _BH


@functools.cache
def _make_sc_gather():
    mesh = plsc.VectorSubcoreMesh(core_axis_name="c", subcore_axis_name="s")

    @functools.partial(
        pl.kernel,
        mesh=mesh,
        out_type=(
            jax.ShapeDtypeStruct((_BH * K, D), jnp.float32),
            jax.ShapeDtypeStruct((_BH * S,), jnp.float32),
            jax.ShapeDtypeStruct((_BH * S,), jnp.float32),
        ),
        scratch_types=[
            pltpu.VMEM((_RPW,), jnp.int32),
            pltpu.VMEM((_PPW,), jnp.int32),
        ] + [pltpu.VMEM((_CHUNK, D), jnp.float32) for _ in range(_NBUF)] + [
            pltpu.VMEM((_PPW,), jnp.float32),
            pltpu.VMEM((_PPW,), jnp.float32),
        ] + [pltpu.SemaphoreType.DMA for _ in range(2 * _NBUF + 1)],
    )
    def sc_gather(table_hbm, idxall_hbm, l0_hbm, l1_hbm,
                  out_hbm, l0s_hbm, l1s_hbm,
                  idxt_v, idxa_v, *rest):
        bufs = rest[:_NBUF]
        l0o_v, l1o_v = rest[_NBUF], rest[_NBUF + 1]
        rsems = rest[_NBUF + 2:2 * _NBUF + 2]
        wsems = rest[2 * _NBUF + 2:3 * _NBUF + 2]
        seml = rest[3 * _NBUF + 2]
        wid = lax.axis_index("s") * _NC + lax.axis_index("c")
        base = wid * _RPW
        pbase = wid * _PPW
        tbase = (wid // _WPR) * S + (wid % _WPR) * _RPW

        pltpu.sync_copy(idxall_hbm.at[pl.ds(tbase, _RPW)], idxt_v)
        rcps = [None] * _NCH
        wcps = [None] * _NCH
        for c in range(min(_NBUF, _NCH)):
            rcps[c] = pltpu.async_copy(
                table_hbm.at[idxt_v.at[pl.ds(c * _CHUNK, _CHUNK)]],
                bufs[c], rsems[c])

        pltpu.sync_copy(idxall_hbm.at[pl.ds(pbase, _PPW)], idxa_v)
        lcps = []
        for q in range(_PPW // 128):
            sl = pl.ds(q * 128, 128)
            lcps.append(pltpu.async_copy(
                l0_hbm.at[idxa_v.at[sl]], l0o_v.at[sl], seml))
            lcps.append(pltpu.async_copy(
                l1_hbm.at[idxa_v.at[sl]], l1o_v.at[sl], seml))

        for c in range(_NCH):
            b = c % _NBUF
            rcps[c].wait()
            wcps[c] = pltpu.async_copy(
                bufs[b], out_hbm.at[pl.ds(base + c * _CHUNK, _CHUNK)],
                wsems[b])
            if c + _NBUF < _NCH:
                wcps[c].wait()
                rcps[c + _NBUF] = pltpu.async_copy(
                    table_hbm.at[idxt_v.at[
                        pl.ds((c + _NBUF) * _CHUNK, _CHUNK)]],
                    bufs[b], rsems[b])
        for c in range(max(0, _NCH - _NBUF), _NCH):
            wcps[c].wait()

        for cp in lcps:
            cp.wait()
        pltpu.sync_copy(l0o_v, l0s_hbm.at[pl.ds(pbase, _PPW)])
        pltpu.sync_copy(l1o_v, l1s_hbm.at[pl.ds(pbase, _PPW)])

    return sc_gather


def kernel(feats, logit):
    mx = jnp.max(logit, axis=-1)
    mn = jnp.min(logit, axis=-1)
    maxp = 1.0 / (1.0 + jnp.exp(mn - mx))
    l0 = logit[..., 0].reshape(B * S)
    l1 = logit[..., 1].reshape(B * S)
    maxp2 = maxp.reshape(B * R, C)
    feats_f = feats.reshape(B * S, D)
    gather = _make_sc_gather()
    sf_hs, p1_hs, p0_hs = [], [], []
    for h in range(B // _BH):
        g2 = _sort_call(maxp2[_BH * R * h:_BH * R * (h + 1)])
        ga = (g2.reshape(_BH, R, C).transpose(0, 2, 1).reshape(_BH * S)
              + _BH * S * h)
        sfh, l0sh, l1sh = gather(feats_f, ga, l0, l1)
        sf_hs.append(sfh.reshape(_BH, K, D))
        l0s = l0sh.reshape(_BH, S)
        l1s = l1sh.reshape(_BH, S)
        p1_hs.append(jnp.stack([l0s[:, :K], l1s[:, :K]], axis=-1))
        p0_hs.append(jnp.stack([l0s[:, K:], l1s[:, K:]], axis=-1))
    return (jnp.concatenate(sf_hs, axis=0),
            jnp.concatenate(p1_hs, axis=0),
            jnp.concatenate(p0_hs, axis=0))


# speedup vs baseline: 1.1934x; 1.1934x over previous
import functools

import jax
import jax.numpy as jnp
from jax import lax
from jax.experimental import pallas as pl
from jax.experimental.pallas import tpu as pltpu
from jax.experimental.pallas import tpu_sc as plsc

B = 4
S = 8192
D = 768
K = 2048
LOG2S = 13


R = 8
C = S // R


def _sort_body(key_ref, gidx_ref):
    key = key_ref[...]
    g = lax.broadcasted_iota(jnp.int32, (B * R, C), 0)
    cc = lax.broadcasted_iota(jnp.int32, (B * R, C), 1)
    it = cc * R + (g & (R - 1))
    idx = (g & (R - 1)) * C + cc

    for klog in range(1, LOG2S + 1):
        kk = 1 << klog
        for jlog in range(klog - 1, -1, -1):
            j = 1 << jlog
            is_hi = (it & j) != 0
            dir_up = (it & kk) == 0

            def partner(x, j=j, is_hi=is_hi):
                if j < R:
                    return jnp.where(is_hi, pltpu.roll(x, j, 0),
                                     pltpu.roll(x, B * R - j, 0))
                d = j // R
                return jnp.where(is_hi, pltpu.roll(x, d, 1),
                                 pltpu.roll(x, C - d, 1))

            pk = partner(key)
            pi = partner(idx)
            less = (key > pk) | ((key == pk) & (idx < pi))
            keep = jnp.logical_xor(less, is_hi) == dir_up
            key = jnp.where(keep, key, pk)
            idx = jnp.where(keep, idx, pi)

    gidx_ref[...] = idx + (g >> 3) * S


_sort_call = pl.pallas_call(
    _sort_body,
    out_shape=jax.ShapeDtypeStruct((B * R, C), jnp.int32),
)


_NC, _NS = 2, 16
_NW = _NC * _NS
_RPW = (B * K) // _NW
_CHUNK = 32
_NBUF = 4
_NCH = _RPW // _CHUNK

_PPW = (B * S) // _NW
_WPR = _NW // B


@functools.cache
def _make_sc_gather():
    mesh = plsc.VectorSubcoreMesh(core_axis_name="c", subcore_axis_name="s")

    @functools.partial(
        pl.kernel,
        mesh=mesh,
        out_type=(
            jax.ShapeDtypeStruct((B * K, D), jnp.float32),
            jax.ShapeDtypeStruct((B * S,), jnp.float32),
            jax.ShapeDtypeStruct((B * S,), jnp.float32),
        ),
        scratch_types=[
            pltpu.VMEM((_RPW,), jnp.int32),
            pltpu.VMEM((_PPW,), jnp.int32),
        ] + [pltpu.VMEM((_CHUNK, D), jnp.float32) for _ in range(_NBUF)] + [
            pltpu.VMEM((_PPW,), jnp.float32),
            pltpu.VMEM((_PPW,), jnp.float32),
        ] + [pltpu.SemaphoreType.DMA for _ in range(2 * _NBUF + 1)],
    )
    def sc_gather(table_hbm, idxall_hbm, l0_hbm, l1_hbm,
                  out_hbm, l0s_hbm, l1s_hbm,
                  idxt_v, idxa_v, *rest):
        bufs = rest[:_NBUF]
        l0o_v, l1o_v = rest[_NBUF], rest[_NBUF + 1]
        rsems = rest[_NBUF + 2:2 * _NBUF + 2]
        wsems = rest[2 * _NBUF + 2:3 * _NBUF + 2]
        seml = rest[3 * _NBUF + 2]
        wid = lax.axis_index("s") * _NC + lax.axis_index("c")
        base = wid * _RPW
        pbase = wid * _PPW
        tbase = (wid // _WPR) * S + (wid % _WPR) * _RPW

        pltpu.sync_copy(idxall_hbm.at[pl.ds(tbase, _RPW)], idxt_v)
        rcps = [None] * _NCH
        wcps = [None] * _NCH
        for c in range(min(_NBUF, _NCH)):
            rcps[c] = pltpu.async_copy(
                table_hbm.at[idxt_v.at[pl.ds(c * _CHUNK, _CHUNK)]],
                bufs[c], rsems[c])

        pltpu.sync_copy(idxall_hbm.at[pl.ds(pbase, _PPW)], idxa_v)
        lcps = []
        for q in range(_PPW // 128):
            sl = pl.ds(q * 128, 128)
            lcps.append(pltpu.async_copy(
                l0_hbm.at[idxa_v.at[sl]], l0o_v.at[sl], seml))
            lcps.append(pltpu.async_copy(
                l1_hbm.at[idxa_v.at[sl]], l1o_v.at[sl], seml))

        for c in range(_NCH):
            b = c % _NBUF
            rcps[c].wait()
            wcps[c] = pltpu.async_copy(
                bufs[b], out_hbm.at[pl.ds(base + c * _CHUNK, _CHUNK)],
                wsems[b])
            if c + _NBUF < _NCH:
                wcps[c].wait()
                rcps[c + _NBUF] = pltpu.async_copy(
                    table_hbm.at[idxt_v.at[
                        pl.ds((c + _NBUF) * _CHUNK, _CHUNK)]],
                    bufs[b], rsems[b])
        for c in range(max(0, _NCH - _NBUF), _NCH):
            wcps[c].wait()

        for cp in lcps:
            cp.wait()
        pltpu.sync_copy(l0o_v, l0s_hbm.at[pl.ds(pbase, _PPW)])
        pltpu.sync_copy(l1o_v, l1s_hbm.at[pl.ds(pbase, _PPW)])

    return sc_gather


def kernel(feats, logit):
    mx = jnp.max(logit, axis=-1)
    mn = jnp.min(logit, axis=-1)
    maxp = 1.0 / (1.0 + jnp.exp(mn - mx))
    l0 = logit[..., 0]
    l1 = logit[..., 1]
    gidx2 = _sort_call(maxp.reshape(B * R, C))
    gidx_all = gidx2.reshape(B, R, C).transpose(0, 2, 1).reshape(B, S)
    sf, l0s_f, l1s_f = _make_sc_gather()(
        feats.reshape(B * S, D), gidx_all.reshape(B * S),
        l0.reshape(B * S), l1.reshape(B * S))
    sf = sf.reshape(B, K, D)
    l0s = l0s_f.reshape(B, S)
    l1s = l1s_f.reshape(B, S)
    preds_1 = jnp.stack([l0s[:, :K], l1s[:, :K]], axis=-1)
    preds_0 = jnp.stack([l0s[:, K:], l1s[:, K:]], axis=-1)
    return sf, preds_1, preds_0
